# baseline (speedup 1.0000x reference)
import functools

import jax
import jax.numpy as jnp
from jax import lax
from jax.experimental import pallas as pl
from jax.experimental.pallas import tpu as pltpu
from jax.experimental.pallas import tpu_sc as plsc

D = 16
B = 16384
F = 26
TOTAL = B * F

_info = plsc.get_sparse_core_info()
_NC, _NS = _info.num_cores, _info.num_subcores
NW = _NC * _NS
PER_W = TOTAL // NW
CHUNK = 1664
NCHUNK = PER_W // CHUNK
assert PER_W % CHUNK == 0 and CHUNK % 8 == 0


def _make_kernel():
    mesh = plsc.VectorSubcoreMesh(core_axis_name="c", subcore_axis_name="s")

    @functools.partial(
        pl.kernel,
        mesh=mesh,
        out_type=jax.ShapeDtypeStruct((TOTAL, D), jnp.float32),
        scratch_types=[
            pltpu.VMEM((CHUNK,), jnp.int32),
            pltpu.VMEM((CHUNK, D), jnp.float32),
            pltpu.SemaphoreType.DMA,
        ],
        compiler_params=pltpu.CompilerParams(use_tc_tiling_on_sc=False),
    )
    def gather_kernel(idx_hbm, table_hbm, out_hbm, idx_v, rows_v, sem):
        wid = lax.axis_index("s") * _NC + lax.axis_index("c")
        base = wid * PER_W
        for c in range(NCHUNK):
            off = base + c * CHUNK
            pltpu.sync_copy(idx_hbm.at[pl.ds(off, CHUNK)], idx_v)
            pltpu.async_copy(table_hbm.at[idx_v], rows_v, sem).wait()
            pltpu.sync_copy(rows_v, out_hbm.at[pl.ds(off, CHUNK)])

    return gather_kernel


_gather = _make_kernel()


@jax.jit
def kernel(x, W):
    idx = x.reshape(-1).astype(jnp.int32)
    out = _gather(idx, W)
    return out.reshape(B, F, D)


# speedup vs baseline: 1.0109x; 1.0109x over previous
import functools

import jax
import jax.numpy as jnp
from jax import lax
from jax.experimental import pallas as pl
from jax.experimental.pallas import tpu as pltpu
from jax.experimental.pallas import tpu_sc as plsc

D = 16
B = 16384
F = 26
TOTAL = B * F

_info = plsc.get_sparse_core_info()
_NC, _NS = _info.num_cores, _info.num_subcores
NW = _NC * _NS
PER_W = TOTAL // NW
CHUNK = 1664
NCHUNK = PER_W // CHUNK
NBUF = 4
assert PER_W % CHUNK == 0 and CHUNK % 8 == 0


def _make_kernel():
    mesh = plsc.VectorSubcoreMesh(core_axis_name="c", subcore_axis_name="s")

    @functools.partial(
        pl.kernel,
        mesh=mesh,
        out_type=jax.ShapeDtypeStruct((TOTAL, D), jnp.float32),
        scratch_types=(
            [pltpu.VMEM((NCHUNK, CHUNK), jnp.int32)]
            + [pltpu.VMEM((CHUNK, D), jnp.float32) for _ in range(NBUF)]
            + [pltpu.SemaphoreType.DMA for _ in range(2 * NBUF)]
        ),
        compiler_params=pltpu.CompilerParams(use_tc_tiling_on_sc=False),
    )
    def gather_kernel(idx_hbm, table_hbm, out_hbm, idx_v, *bufs):
        rows = bufs[:NBUF]
        gsem = bufs[NBUF:2 * NBUF]
        osem = bufs[2 * NBUF:]
        wid = lax.axis_index("s") * _NC + lax.axis_index("c")
        base = wid * PER_W
        pltpu.sync_copy(idx_hbm.at[pl.ds(wid * NCHUNK, NCHUNK)], idx_v)

        def start_gather(c):
            p = c % NBUF
            return pltpu.async_copy(table_hbm.at[idx_v.at[c]], rows[p], gsem[p])

        def start_out(c):
            p = c % NBUF
            return pltpu.async_copy(
                rows[p], out_hbm.at[pl.ds(base + c * CHUNK, CHUNK)], osem[p])

        gathers = [start_gather(c) for c in range(NBUF)]
        outs = [None] * NCHUNK
        for c in range(NCHUNK):
            p = c % NBUF
            gathers[c].wait()
            outs[c] = start_out(c)
            nxt = c + NBUF
            if nxt < NCHUNK:
                outs[c].wait()
                gathers.append(start_gather(nxt))
        for c in range(NCHUNK - NBUF, NCHUNK):
            if outs[c] is not None:
                outs[c].wait()

    return gather_kernel


_gather = _make_kernel()


@jax.jit
def kernel(x, W):
    idx = x.reshape(NW * NCHUNK, CHUNK).astype(jnp.int32)
    out = _gather(idx, W)
    return out.reshape(B, F, D)
